# Initial kernel scaffold; baseline (speedup 1.0000x reference)
import jax
import jax.numpy as jnp
from jax.experimental import pallas as pl


def kernel(x_tmp, edge_index, conv1_init_w, conv1_root_w, conv1_bias, conv2_init_w, conv2_root_w, conv2_bias, fc2_w, fc2_b):
    raise NotImplementedError("write your pallas kernel here")


# speedup vs baseline: 13.6256x; 13.6256x over previous
import functools

import jax
import jax.numpy as jnp
from jax import lax
from jax.experimental import pallas as pl
from jax.experimental.pallas import tpu as pltpu
from jax.experimental.pallas import tpu_sc as plsc

NC = 2
NS = 16
NW = NC * NS
CH = 128

_F32 = jnp.float32
_I32 = jnp.int32


def _mesh():
    return plsc.VectorSubcoreMesh(core_axis_name="c", subcore_axis_name="s")


def _sc_scatter_scalar(src, dst, vals_hbm, zeros1, ones, n_pad):
    e = dst.shape[0]
    ew = e // NW
    nfull = ew // CH
    tail = ew - nfull * CH
    npt = n_pad // NS
    gather = vals_hbm is not None

    scratch = [
        pltpu.VMEM((CH,), _I32),
        pltpu.VMEM((CH,), _F32),
        pltpu.VMEM((tail,), _I32),
        pltpu.VMEM((tail,), _F32),
        pltpu.VMEM((npt,), _F32),
        pltpu.VMEM_SHARED((n_pad,), _F32),
        pltpu.SemaphoreType.DMA,
    ]
    if gather:
        scratch = [pltpu.VMEM((CH,), _I32), pltpu.VMEM((tail,), _I32)] + scratch

    @functools.partial(
        pl.kernel,
        out_type=jax.ShapeDtypeStruct((NC * n_pad,), _F32),
        mesh=_mesh(),
        scratch_types=scratch,
    )
    def k(*refs):
        if gather:
            src_h, dst_h, val_h, z_h, ones_h, out_h, sidx_v, sidxt_v, \
                didx_v, val_v, didxt_v, valt_v, bb_v, acc, sem = refs
        else:
            dst_h, z_h, ones_h, out_h, didx_v, val_v, didxt_v, valt_v, \
                bb_v, acc, sem = refs
        c = lax.axis_index("c")
        s = lax.axis_index("s")
        wid = c * NS + s
        moff = pl.multiple_of(s * npt, 8)
        pltpu.sync_copy(z_h.at[pl.ds(moff, npt)], bb_v)
        pltpu.sync_copy(bb_v, acc.at[pl.ds(moff, npt)])
        if not gather:
            pltpu.sync_copy(ones_h, val_v)
            pltpu.sync_copy(ones_h.at[pl.ds(0, tail)], valt_v)
        plsc.subcore_barrier()
        base = wid * ew

        def body(i, carry):
            off = pl.multiple_of(base + i * CH, 8)
            pltpu.sync_copy(dst_h.at[pl.ds(off, CH)], didx_v)
            if gather:
                pltpu.sync_copy(src_h.at[pl.ds(off, CH)], sidx_v)
                pltpu.async_copy(val_h.at[sidx_v], val_v, sem).wait()
            pltpu.sync_copy(val_v, acc.at[didx_v], add=True)
            return carry

        lax.fori_loop(0, nfull, body, 0)
        off = pl.multiple_of(base + nfull * CH, 8)
        pltpu.sync_copy(dst_h.at[pl.ds(off, tail)], didxt_v)
        if gather:
            pltpu.sync_copy(src_h.at[pl.ds(off, tail)], sidxt_v)
            pltpu.async_copy(val_h.at[sidxt_v], valt_v, sem).wait()
        pltpu.sync_copy(valt_v, acc.at[didxt_v], add=True)
        plsc.subcore_barrier()
        ooff = pl.multiple_of(c * n_pad + s * npt, 8)
        pltpu.sync_copy(acc.at[pl.ds(moff, npt)], bb_v)
        pltpu.sync_copy(bb_v, out_h.at[pl.ds(ooff, npt)])

    if gather:
        return k(src, dst, vals_hbm, zeros1, ones)
    return k(dst, zeros1, ones)


def _sc_scatter_rows(src, dst, rows0, rows1, zeros2, n_pad, dh):
    e = dst.shape[0]
    et = e // NS
    nfull = et // CH
    tail = et - nfull * CH
    npt = n_pad // NS

    @functools.partial(
        pl.kernel,
        out_type=[jax.ShapeDtypeStruct((n_pad, dh), _F32)] * 2,
        mesh=_mesh(),
        compiler_params=pltpu.CompilerParams(use_tc_tiling_on_sc=False),
        scratch_types=[
            pltpu.VMEM((CH,), _I32),
            pltpu.VMEM((CH,), _I32),
            pltpu.VMEM((CH, dh), _F32),
            pltpu.VMEM((tail,), _I32),
            pltpu.VMEM((tail,), _I32),
            pltpu.VMEM((tail, dh), _F32),
            pltpu.VMEM((npt, dh), _F32),
            pltpu.VMEM_SHARED((n_pad, dh), _F32),
            pltpu.SemaphoreType.DMA,
        ],
    )
    def k(src_h, dst_h, rows0_h, rows1_h, z_h, out0_h, out1_h,
          sidx_v, didx_v, rows_v, sidxt_v, didxt_v, rowst_v, bb_v,
          acc, sem):
        c = lax.axis_index("c")
        s = lax.axis_index("s")
        moff = pl.multiple_of(s * npt, 8)
        pltpu.sync_copy(z_h.at[pl.ds(moff, npt), :], bb_v)
        pltpu.sync_copy(bb_v, acc.at[pl.ds(moff, npt), :])
        plsc.subcore_barrier()
        base = s * et

        def body(i, carry):
            off = pl.multiple_of(base + i * CH, 8)
            pltpu.sync_copy(src_h.at[pl.ds(off, CH)], sidx_v)
            pltpu.sync_copy(dst_h.at[pl.ds(off, CH)], didx_v)

            @pl.when(c == 0)
            def _():
                pltpu.async_copy(rows0_h.at[sidx_v], rows_v, sem).wait()

            @pl.when(c == 1)
            def _():
                pltpu.async_copy(rows1_h.at[sidx_v], rows_v, sem).wait()

            pltpu.sync_copy(rows_v, acc.at[didx_v], add=True)
            return carry

        lax.fori_loop(0, nfull, body, 0)
        off = pl.multiple_of(base + nfull * CH, 8)
        pltpu.sync_copy(src_h.at[pl.ds(off, tail)], sidxt_v)
        pltpu.sync_copy(dst_h.at[pl.ds(off, tail)], didxt_v)

        @pl.when(c == 0)
        def _():
            pltpu.async_copy(rows0_h.at[sidxt_v], rowst_v, sem).wait()

        @pl.when(c == 1)
        def _():
            pltpu.async_copy(rows1_h.at[sidxt_v], rowst_v, sem).wait()

        pltpu.sync_copy(rowst_v, acc.at[didxt_v], add=True)
        plsc.subcore_barrier()
        pltpu.sync_copy(acc.at[pl.ds(moff, npt), :], bb_v)

        @pl.when(c == 0)
        def _():
            pltpu.sync_copy(bb_v, out0_h.at[pl.ds(moff, npt), :])

        @pl.when(c == 1)
        def _():
            pltpu.sync_copy(bb_v, out1_h.at[pl.ds(moff, npt), :])

    return k(src, dst, rows0, rows1, zeros2)


def _tc_layer1(svec, xp, dis, w1i, w1r, b1, n_pad, blk):
    d = w1i.shape[1]

    dh = d // NC

    def body(s_r, x_r, d_r, wi_r, wr_r, b_r, h_r, hh0_r, hh1_r):
        h = jnp.maximum(s_r[:] * wi_r[:] + x_r[:] * wr_r[:] + b_r[:], 0.0)
        h_r[:] = h
        hh = h * d_r[:]
        hh0_r[:] = hh[:, :dh]
        hh1_r[:] = hh[:, dh:]

    col = pl.BlockSpec((blk, 1), lambda i: (i, 0))
    full = pl.BlockSpec((1, d), lambda i: (0, 0))
    return pl.pallas_call(
        body,
        grid=(n_pad // blk,),
        in_specs=[col, col, col, full, full, full],
        out_specs=[pl.BlockSpec((blk, d), lambda i: (i, 0)),
                   pl.BlockSpec((blk, dh), lambda i: (i, 0)),
                   pl.BlockSpec((blk, dh), lambda i: (i, 0))],
        out_shape=[jax.ShapeDtypeStruct((n_pad, d), _F32),
                   jax.ShapeDtypeStruct((n_pad, dh), _F32),
                   jax.ShapeDtypeStruct((n_pad, dh), _F32)],
    )(svec, xp, dis, w1i, w1r, b1)


def _tc_layer2(u0, u1, dis, h1, w2i, w2r, b2, fw, fb, n_pad, blk):
    def body(u0_r, u1_r, d_r, h1_r, wi_r, wr_r, b_r, fw_r, fb_r, y_r):
        g = jnp.concatenate([u0_r[:], u1_r[:]], axis=1) * d_r[:]
        agg = jnp.dot(g, wi_r[:], preferred_element_type=_F32)
        rt = jnp.dot(h1_r[:], wr_r[:], preferred_element_type=_F32)
        out2 = jnp.maximum(agg + rt + b_r[:], 0.0)
        y_r[:] = jnp.dot(out2, fw_r[:], preferred_element_type=_F32) + fb_r[:]

    dh = h1.shape[1]
    dhh = dh // NC
    do = w2i.shape[1]
    wide = pl.BlockSpec((blk, dh), lambda i: (i, 0))
    half = pl.BlockSpec((blk, dhh), lambda i: (i, 0))
    col = pl.BlockSpec((blk, 1), lambda i: (i, 0))
    return pl.pallas_call(
        body,
        grid=(n_pad // blk,),
        in_specs=[half, half, col, wide,
                  pl.BlockSpec((dh, do), lambda i: (0, 0)),
                  pl.BlockSpec((dh, do), lambda i: (0, 0)),
                  pl.BlockSpec((1, do), lambda i: (0, 0)),
                  pl.BlockSpec((do, 1), lambda i: (0, 0)),
                  pl.BlockSpec((1, 1), lambda i: (0, 0))],
        out_specs=pl.BlockSpec((blk, 1), lambda i: (i, 0)),
        out_shape=jax.ShapeDtypeStruct((n_pad, 1), _F32),
    )(u0, u1, dis, h1, w2i, w2r, b2, fw, fb)


def kernel(x_tmp, edge_index, conv1_init_w, conv1_root_w, conv1_bias,
           conv2_init_w, conv2_root_w, conv2_bias, fc2_w, fc2_b):
    n = x_tmp.shape[0]
    n_pad = ((n + 255) // 256) * 256
    blk = n_pad // 16

    src = edge_index[0]
    dst = edge_index[1]
    zeros1 = jnp.zeros((n_pad,), _F32)
    ones = jnp.ones((CH,), _F32)

    deg2 = _sc_scatter_scalar(None, dst, None, zeros1, ones, n_pad)
    deg = deg2[:n_pad] + deg2[n_pad:]
    dis = jnp.where(deg > 0, lax.rsqrt(jnp.maximum(deg, 1e-12)), 0.0)
    xp = jnp.pad(x_tmp[:, 0], (0, n_pad - n))
    xhat = dis * xp

    t2 = _sc_scatter_scalar(src, dst, xhat, zeros1, ones, n_pad)
    svec = dis * (t2[:n_pad] + t2[n_pad:])

    h1, hh0, hh1 = _tc_layer1(svec[:, None], xp[:, None], dis[:, None],
                              conv1_init_w, conv1_root_w, conv1_bias[None, :],
                              n_pad, blk)

    dhh = hh0.shape[1]
    zeros2 = jnp.zeros((n_pad, dhh), _F32)
    u0, u1 = _sc_scatter_rows(src, dst, hh0, hh1, zeros2, n_pad, dhh)

    y = _tc_layer2(u0, u1, dis[:, None], h1,
                   conv2_init_w, conv2_root_w, conv2_bias[None, :],
                   fc2_w, fc2_b[None, :], n_pad, blk)
    return y[:n]


# speedup vs baseline: 34.2616x; 2.5145x over previous
import functools

import jax
import jax.numpy as jnp
from jax import lax
from jax.experimental import pallas as pl
from jax.experimental.pallas import tpu as pltpu
from jax.experimental.pallas import tpu_sc as plsc

NC = 2
NS = 16
NW = NC * NS
CH = 128

_F32 = jnp.float32
_I32 = jnp.int32

_SC_PARAMS = dict(
    mesh=plsc.VectorSubcoreMesh(core_axis_name="c", subcore_axis_name="s"),
    compiler_params=pltpu.CompilerParams(use_tc_tiling_on_sc=False),
)


def _sc_scatter_scalar(src2, dst2, vals_hbm, zeros1, ones, n_pad, kk):
    rows = dst2.shape[0]
    rw = rows // NW
    nsup = rw // kk
    assert nsup * kk == rw
    npt = n_pad // NS
    gather = vals_hbm is not None

    scratch = [
        pltpu.VMEM((kk, CH), _I32),
        pltpu.VMEM((kk, CH), _F32),
        pltpu.VMEM((npt,), _F32),
        pltpu.VMEM_SHARED((n_pad,), _F32),
        pltpu.SemaphoreType.DMA,
        pltpu.SemaphoreType.DMA,
    ]
    if gather:
        scratch = [pltpu.VMEM((kk, CH), _I32)] + scratch

    @functools.partial(
        pl.kernel,
        out_type=jax.ShapeDtypeStruct((NC * n_pad,), _F32),
        scratch_types=scratch,
        **_SC_PARAMS,
    )
    def k(*refs):
        if gather:
            src_h, dst_h, val_h, z_h, out_h, sidx_v, didx_v, val_v, \
                bb_v, acc, semg, sems = refs
        else:
            dst_h, z_h, ones_h, out_h, didx_v, val_v, bb_v, acc, semg, sems = refs
        c = lax.axis_index("c")
        s = lax.axis_index("s")
        wid = c * NS + s
        moff = pl.multiple_of(s * npt, 8)
        pltpu.sync_copy(z_h.at[pl.ds(moff, npt)], bb_v)
        pltpu.sync_copy(bb_v, acc.at[pl.ds(moff, npt)])
        if not gather:
            pltpu.sync_copy(ones_h, val_v)
        plsc.subcore_barrier()
        base = wid * rw

        def body(i, carry):
            r0 = pl.multiple_of(base + i * kk, 8)
            pltpu.sync_copy(dst_h.at[pl.ds(r0, kk), :], didx_v)
            if gather:
                pltpu.sync_copy(src_h.at[pl.ds(r0, kk), :], sidx_v)
                gd = [pltpu.async_copy(val_h.at[sidx_v.at[j]], val_v.at[j], semg)
                      for j in range(kk)]
                for d_ in gd:
                    d_.wait()
            sd = [pltpu.async_copy(val_v.at[j], acc.at[didx_v.at[j]], sems,
                                   add=True)
                  for j in range(kk)]
            for d_ in sd:
                d_.wait()
            return carry

        lax.fori_loop(0, nsup, body, 0)
        plsc.subcore_barrier()
        ooff = pl.multiple_of(c * n_pad + s * npt, 8)
        pltpu.sync_copy(acc.at[pl.ds(moff, npt)], bb_v)
        pltpu.sync_copy(bb_v, out_h.at[pl.ds(ooff, npt)])

    if gather:
        return k(src2, dst2, vals_hbm, zeros1)
    return k(dst2, zeros1, ones)


def _sc_scatter_rows(src2, dst2, rows0, rows1, zeros2, n_pad, dh, kk):
    rows = dst2.shape[0]
    rt = rows // NS
    nsup = rt // kk
    assert nsup * kk == rt
    npt = n_pad // NS

    @functools.partial(
        pl.kernel,
        out_type=[jax.ShapeDtypeStruct((n_pad, dh), _F32)] * 2,
        scratch_types=[
            pltpu.VMEM((kk, CH), _I32),
            pltpu.VMEM((kk, CH), _I32),
            pltpu.VMEM((kk, CH, dh), _F32),
            pltpu.VMEM((npt, dh), _F32),
            pltpu.VMEM_SHARED((n_pad, dh), _F32),
            pltpu.SemaphoreType.DMA,
            pltpu.SemaphoreType.DMA,
        ],
        **_SC_PARAMS,
    )
    def k(src_h, dst_h, rows0_h, rows1_h, z_h, out0_h, out1_h,
          sidx_v, didx_v, rows_v, bb_v, acc, semg, sems):
        c = lax.axis_index("c")
        s = lax.axis_index("s")
        moff = pl.multiple_of(s * npt, 8)
        pltpu.sync_copy(z_h.at[pl.ds(moff, npt), :], bb_v)
        pltpu.sync_copy(bb_v, acc.at[pl.ds(moff, npt), :])
        plsc.subcore_barrier()
        base = s * rt

        def body(i, carry):
            r0 = pl.multiple_of(base + i * kk, 8)
            pltpu.sync_copy(src_h.at[pl.ds(r0, kk), :], sidx_v)
            pltpu.sync_copy(dst_h.at[pl.ds(r0, kk), :], didx_v)

            @pl.when(c == 0)
            def _():
                gd = [pltpu.async_copy(rows0_h.at[sidx_v.at[j]], rows_v.at[j],
                                       semg)
                      for j in range(kk)]
                for d_ in gd:
                    d_.wait()

            @pl.when(c == 1)
            def _():
                gd = [pltpu.async_copy(rows1_h.at[sidx_v.at[j]], rows_v.at[j],
                                       semg)
                      for j in range(kk)]
                for d_ in gd:
                    d_.wait()

            sd = [pltpu.async_copy(rows_v.at[j], acc.at[didx_v.at[j]], sems,
                                   add=True)
                  for j in range(kk)]
            for d_ in sd:
                d_.wait()
            return carry

        lax.fori_loop(0, nsup, body, 0)
        plsc.subcore_barrier()
        pltpu.sync_copy(acc.at[pl.ds(moff, npt), :], bb_v)

        @pl.when(c == 0)
        def _():
            pltpu.sync_copy(bb_v, out0_h.at[pl.ds(moff, npt), :])

        @pl.when(c == 1)
        def _():
            pltpu.sync_copy(bb_v, out1_h.at[pl.ds(moff, npt), :])

    return k(src2, dst2, rows0, rows1, zeros2)


def _tc_layer1(svec, xp, dis, w1i, w1r, b1, n_pad, blk):
    d = w1i.shape[1]
    dh = d // NC

    def body(s_r, x_r, d_r, wi_r, wr_r, b_r, h_r, hh0_r, hh1_r):
        h = jnp.maximum(s_r[:] * wi_r[:] + x_r[:] * wr_r[:] + b_r[:], 0.0)
        h_r[:] = h
        hh = h * d_r[:]
        hh0_r[:] = hh[:, :dh]
        hh1_r[:] = hh[:, dh:]

    col = pl.BlockSpec((blk, 1), lambda i: (i, 0))
    full = pl.BlockSpec((1, d), lambda i: (0, 0))
    return pl.pallas_call(
        body,
        grid=(n_pad // blk,),
        in_specs=[col, col, col, full, full, full],
        out_specs=[pl.BlockSpec((blk, d), lambda i: (i, 0)),
                   pl.BlockSpec((blk, dh), lambda i: (i, 0)),
                   pl.BlockSpec((blk, dh), lambda i: (i, 0))],
        out_shape=[jax.ShapeDtypeStruct((n_pad, d), _F32),
                   jax.ShapeDtypeStruct((n_pad, dh), _F32),
                   jax.ShapeDtypeStruct((n_pad, dh), _F32)],
    )(svec, xp, dis, w1i, w1r, b1)


def _tc_layer2(u0, u1, dis, h1, w2i, w2r, b2, fw, fb, n_pad, blk):
    def body(u0_r, u1_r, d_r, h1_r, wi_r, wr_r, b_r, fw_r, fb_r, y_r):
        g = jnp.concatenate([u0_r[:], u1_r[:]], axis=1) * d_r[:]
        agg = jnp.dot(g, wi_r[:], preferred_element_type=_F32)
        rt = jnp.dot(h1_r[:], wr_r[:], preferred_element_type=_F32)
        out2 = jnp.maximum(agg + rt + b_r[:], 0.0)
        y_r[:] = jnp.dot(out2, fw_r[:], preferred_element_type=_F32) + fb_r[:]

    dh = h1.shape[1]
    dhh = dh // NC
    do = w2i.shape[1]
    wide = pl.BlockSpec((blk, dh), lambda i: (i, 0))
    half = pl.BlockSpec((blk, dhh), lambda i: (i, 0))
    col = pl.BlockSpec((blk, 1), lambda i: (i, 0))
    return pl.pallas_call(
        body,
        grid=(n_pad // blk,),
        in_specs=[half, half, col, wide,
                  pl.BlockSpec((dh, do), lambda i: (0, 0)),
                  pl.BlockSpec((dh, do), lambda i: (0, 0)),
                  pl.BlockSpec((1, do), lambda i: (0, 0)),
                  pl.BlockSpec((do, 1), lambda i: (0, 0)),
                  pl.BlockSpec((1, 1), lambda i: (0, 0))],
        out_specs=pl.BlockSpec((blk, 1), lambda i: (i, 0)),
        out_shape=jax.ShapeDtypeStruct((n_pad, 1), _F32),
    )(u0, u1, dis, h1, w2i, w2r, b2, fw, fb)


def kernel(x_tmp, edge_index, conv1_init_w, conv1_root_w, conv1_bias,
           conv2_init_w, conv2_root_w, conv2_bias, fc2_w, fc2_b):
    n = x_tmp.shape[0]
    e = edge_index.shape[1]
    n_pad = ((n + 255) // 256) * 256
    blk = n_pad // 16

    kk_s, kk_r = 8, 8
    row_quant = CH * kk_s * NW
    e_pad = ((e + row_quant - 1) // row_quant) * row_quant
    npad_e = e_pad - e
    rows = e_pad // CH
    pad_ar = jnp.arange(npad_e, dtype=_I32)
    src2 = jnp.concatenate([edge_index[0], pad_ar % n]).reshape(rows, CH)
    dst2 = jnp.concatenate([edge_index[1], n + pad_ar % (n_pad - n)]
                           ).reshape(rows, CH)

    zeros1 = jnp.zeros((n_pad,), _F32)
    ones = jnp.ones((kk_s, CH), _F32)

    deg2 = _sc_scatter_scalar(None, dst2, None, zeros1, ones, n_pad, kk_s)
    deg = deg2[:n_pad] + deg2[n_pad:]
    dis = jnp.where(deg > 0, lax.rsqrt(jnp.maximum(deg, 1e-12)), 0.0)
    xp = jnp.pad(x_tmp[:, 0], (0, n_pad - n))
    xhat = dis * xp

    t2 = _sc_scatter_scalar(src2, dst2, xhat, zeros1, ones, n_pad, kk_s)
    svec = dis * (t2[:n_pad] + t2[n_pad:])

    h1, hh0, hh1 = _tc_layer1(svec[:, None], xp[:, None], dis[:, None],
                              conv1_init_w, conv1_root_w, conv1_bias[None, :],
                              n_pad, blk)

    dhh = hh0.shape[1]
    zeros2 = jnp.zeros((n_pad, dhh), _F32)
    u0, u1 = _sc_scatter_rows(src2, dst2, hh0, hh1, zeros2, n_pad, dhh, kk_r)

    y = _tc_layer2(u0, u1, dis[:, None], h1,
                   conv2_init_w, conv2_root_w, conv2_bias[None, :],
                   fc2_w, fc2_b[None, :], n_pad, blk)
    return y[:n]


# speedup vs baseline: 44.4719x; 1.2980x over previous
import functools

import jax
import jax.numpy as jnp
from jax import lax
from jax.experimental import pallas as pl
from jax.experimental.pallas import tpu as pltpu
from jax.experimental.pallas import tpu_sc as plsc

NC = 2
NS = 16
NW = NC * NS
CH = 128
LN = 16

_F32 = jnp.float32
_I32 = jnp.int32

_SC_PARAMS = dict(
    mesh=plsc.VectorSubcoreMesh(core_axis_name="c", subcore_axis_name="s"),
    compiler_params=pltpu.CompilerParams(use_tc_tiling_on_sc=False, needs_layout_passes=False),
)


def _sc_scatter_scalar(src2, dst2, vals_hbm, zeros1, ones, n_pad, kk):
    rows = dst2.shape[0]
    rw = rows // NW
    nsup = rw // kk
    assert nsup * kk == rw
    npt = n_pad // NS
    gather = vals_hbm is not None

    scratch = [
        pltpu.VMEM((kk, CH), _I32),
        pltpu.VMEM((kk, CH), _F32),
        pltpu.VMEM((npt,), _F32),
        pltpu.VMEM_SHARED((n_pad,), _F32),
        pltpu.SemaphoreType.DMA,
    ]
    if gather:
        scratch = [pltpu.VMEM((kk, CH), _I32),
                   pltpu.VMEM((n_pad,), _F32)] + scratch

    @functools.partial(
        pl.kernel,
        out_type=jax.ShapeDtypeStruct((NC * n_pad,), _F32),
        scratch_types=scratch,
        **_SC_PARAMS,
    )
    def k(*refs):
        if gather:
            src_h, dst_h, val_h, z_h, out_h, sidx_v, xtab_v, didx_v, val_v, \
                bb_v, acc, sems = refs
        else:
            dst_h, z_h, ones_h, out_h, didx_v, val_v, bb_v, acc, sems = refs
        c = lax.axis_index("c")
        s = lax.axis_index("s")
        wid = c * NS + s
        moff = pl.multiple_of(s * npt, 8)
        pltpu.sync_copy(z_h.at[pl.ds(moff, npt)], bb_v)
        pltpu.sync_copy(bb_v, acc.at[pl.ds(moff, npt)])
        if gather:
            pltpu.sync_copy(val_h, xtab_v)
        else:
            pltpu.sync_copy(ones_h, val_v)
        plsc.subcore_barrier()
        base = wid * rw

        def body(i, carry):
            r0 = pl.multiple_of(base + i * kk, 8)
            pltpu.sync_copy(dst_h.at[pl.ds(r0, kk), :], didx_v)
            if gather:
                pltpu.sync_copy(src_h.at[pl.ds(r0, kk), :], sidx_v)
                for j in range(kk):
                    for q in range(CH // LN):
                        idx16 = sidx_v[j, pl.ds(q * LN, LN)]
                        val_v[j, pl.ds(q * LN, LN)] = \
                            plsc.load_gather(xtab_v, [idx16])
            sd = [pltpu.async_copy(val_v.at[j], acc.at[didx_v.at[j]], sems,
                                   add=True)
                  for j in range(kk)]
            for d_ in sd:
                d_.wait()
            return carry

        lax.fori_loop(0, nsup, body, 0)
        plsc.subcore_barrier()
        ooff = pl.multiple_of(c * n_pad + s * npt, 8)
        pltpu.sync_copy(acc.at[pl.ds(moff, npt)], bb_v)
        pltpu.sync_copy(bb_v, out_h.at[pl.ds(ooff, npt)])

    if gather:
        return k(src2, dst2, vals_hbm, zeros1)
    return k(dst2, zeros1, ones)


def _sc_scatter_rows(src2, dst2, rows0, rows1, zeros2, n_pad, dh, kk):
    rows = dst2.shape[0]
    rt = rows // NS
    nsup = rt // kk
    assert nsup * kk == rt
    npt = n_pad // NS

    @functools.partial(
        pl.kernel,
        out_type=[jax.ShapeDtypeStruct((n_pad, dh), _F32)] * 2,
        scratch_types=[
            pltpu.VMEM((kk, CH), _I32),
            pltpu.VMEM((kk, CH), _I32),
            pltpu.VMEM((kk, CH, dh), _F32),
            pltpu.VMEM((npt, dh), _F32),
            pltpu.VMEM_SHARED((n_pad, dh), _F32),
            pltpu.SemaphoreType.DMA,
            pltpu.SemaphoreType.DMA,
        ],
        **_SC_PARAMS,
    )
    def k(src_h, dst_h, rows0_h, rows1_h, z_h, out0_h, out1_h,
          sidx_v, didx_v, rows_v, bb_v, acc, semg, sems):
        c = lax.axis_index("c")
        s = lax.axis_index("s")
        moff = pl.multiple_of(s * npt, 8)
        pltpu.sync_copy(z_h.at[pl.ds(moff, npt), :], bb_v)
        pltpu.sync_copy(bb_v, acc.at[pl.ds(moff, npt), :])
        plsc.subcore_barrier()
        base = s * rt

        def body(i, carry):
            r0 = pl.multiple_of(base + i * kk, 8)
            pltpu.sync_copy(src_h.at[pl.ds(r0, kk), :], sidx_v)
            pltpu.sync_copy(dst_h.at[pl.ds(r0, kk), :], didx_v)

            def fire_scatters(gd):
                sd = []
                for j in range(kk):
                    gd[j].wait()
                    sd.append(pltpu.async_copy(rows_v.at[j],
                                               acc.at[didx_v.at[j]],
                                               sems, add=True))
                for d_ in sd:
                    d_.wait()

            @pl.when(c == 0)
            def _():
                fire_scatters([pltpu.async_copy(rows0_h.at[sidx_v.at[j]],
                                                rows_v.at[j], semg)
                               for j in range(kk)])

            @pl.when(c == 1)
            def _():
                fire_scatters([pltpu.async_copy(rows1_h.at[sidx_v.at[j]],
                                                rows_v.at[j], semg)
                               for j in range(kk)])

            return carry

        lax.fori_loop(0, nsup, body, 0)
        plsc.subcore_barrier()
        pltpu.sync_copy(acc.at[pl.ds(moff, npt), :], bb_v)

        @pl.when(c == 0)
        def _():
            pltpu.sync_copy(bb_v, out0_h.at[pl.ds(moff, npt), :])

        @pl.when(c == 1)
        def _():
            pltpu.sync_copy(bb_v, out1_h.at[pl.ds(moff, npt), :])

    return k(src2, dst2, rows0, rows1, zeros2)


def _tc_layer1(svec, xp, dis, w1iT, w1rT, b1T, n_pad, blk):
    d = w1iT.shape[0]
    dh = d // NC

    def body(s_r, x_r, d_r, wi_r, wr_r, b_r, h_r, hh0_r, hh1_r):
        h = jnp.maximum(wi_r[:] * s_r[:] + wr_r[:] * x_r[:] + b_r[:], 0.0)
        h_r[:] = h
        hh = h * d_r[:]
        hh0_r[:] = hh[:dh, :]
        hh1_r[:] = hh[dh:, :]

    row = pl.BlockSpec((1, blk), lambda i: (0, i))
    full = pl.BlockSpec((d, 1), lambda i: (0, 0))
    return pl.pallas_call(
        body,
        grid=(n_pad // blk,),
        in_specs=[row, row, row, full, full, full],
        out_specs=[pl.BlockSpec((d, blk), lambda i: (0, i)),
                   pl.BlockSpec((dh, blk), lambda i: (0, i)),
                   pl.BlockSpec((dh, blk), lambda i: (0, i))],
        out_shape=[jax.ShapeDtypeStruct((d, n_pad), _F32),
                   jax.ShapeDtypeStruct((dh, n_pad), _F32),
                   jax.ShapeDtypeStruct((dh, n_pad), _F32)],
    )(svec, xp, dis, w1iT, w1rT, b1T)


def _tc_layer2(u0T, u1T, dis, h1T, w2iT, w2rT, b2T, fwT, fb, n_pad, blk):
    def body(u0_r, u1_r, d_r, h1_r, wi_r, wr_r, b_r, fw_r, fb_r, y_r):
        g = jnp.concatenate([u0_r[:], u1_r[:]], axis=0) * d_r[:]
        agg = jnp.dot(wi_r[:], g, preferred_element_type=_F32)
        rt = jnp.dot(wr_r[:], h1_r[:], preferred_element_type=_F32)
        out2 = jnp.maximum(agg + rt + b_r[:], 0.0)
        y_r[:] = jnp.dot(fw_r[:], out2, preferred_element_type=_F32) + fb_r[:]

    dd = h1T.shape[0]
    dhh = dd // NC
    do = w2iT.shape[0]
    row = pl.BlockSpec((1, blk), lambda i: (0, i))
    half = pl.BlockSpec((dhh, blk), lambda i: (0, i))
    return pl.pallas_call(
        body,
        grid=(n_pad // blk,),
        in_specs=[half, half, row, pl.BlockSpec((dd, blk), lambda i: (0, i)),
                  pl.BlockSpec((do, dd), lambda i: (0, 0)),
                  pl.BlockSpec((do, dd), lambda i: (0, 0)),
                  pl.BlockSpec((do, 1), lambda i: (0, 0)),
                  pl.BlockSpec((1, do), lambda i: (0, 0)),
                  pl.BlockSpec((1, 1), lambda i: (0, 0))],
        out_specs=pl.BlockSpec((1, blk), lambda i: (0, i)),
        out_shape=jax.ShapeDtypeStruct((1, n_pad), _F32),
    )(u0T, u1T, dis, h1T, w2iT, w2rT, b2T, fwT, fb)


def kernel(x_tmp, edge_index, conv1_init_w, conv1_root_w, conv1_bias,
           conv2_init_w, conv2_root_w, conv2_bias, fc2_w, fc2_b):
    n = x_tmp.shape[0]
    e = edge_index.shape[1]
    n_pad = ((n + 255) // 256) * 256
    blk = n_pad // 4

    kk_s, kk_r = 8, 8
    row_quant = CH * kk_s * NW
    e_pad = ((e + row_quant - 1) // row_quant) * row_quant
    npad_e = e_pad - e
    rows = e_pad // CH
    pad_ar = jnp.arange(npad_e, dtype=_I32)
    src2 = jnp.concatenate([edge_index[0], pad_ar % n]).reshape(rows, CH)
    dst2 = jnp.concatenate([edge_index[1], n + pad_ar % (n_pad - n)]
                           ).reshape(rows, CH)

    zeros1 = jnp.zeros((n_pad,), _F32)
    ones = jnp.ones((kk_s, CH), _F32)

    deg2 = _sc_scatter_scalar(None, dst2, None, zeros1, ones, n_pad, kk_s)
    deg = deg2[:n_pad] + deg2[n_pad:]
    dis = jnp.where(deg > 0, lax.rsqrt(jnp.maximum(deg, 1e-12)), 0.0)
    xp = jnp.pad(x_tmp[:, 0], (0, n_pad - n))
    xhat = dis * xp

    t2 = _sc_scatter_scalar(src2, dst2, xhat, zeros1, ones, n_pad, kk_s)
    svec = dis * (t2[:n_pad] + t2[n_pad:])

    h1T, hh0T, hh1T = _tc_layer1(
        svec[None, :], xp[None, :], dis[None, :],
        conv1_init_w.reshape(-1, 1), conv1_root_w.reshape(-1, 1),
        conv1_bias[:, None], n_pad, blk)

    dhh = hh0T.shape[0]
    zeros2 = jnp.zeros((n_pad, dhh), _F32)
    u0, u1 = _sc_scatter_rows(src2, dst2, hh0T.T, hh1T.T, zeros2,
                              n_pad, dhh, kk_r)

    yT = _tc_layer2(u0.T, u1.T, dis[None, :], h1T,
                    conv2_init_w.T, conv2_root_w.T, conv2_bias[:, None],
                    fc2_w.T, fc2_b[None, :], n_pad, blk)
    return yT[0, :n][:, None]


# speedup vs baseline: 47.0384x; 1.0577x over previous
import functools

import jax
import jax.numpy as jnp
from jax import lax
from jax.experimental import pallas as pl
from jax.experimental.pallas import tpu as pltpu
from jax.experimental.pallas import tpu_sc as plsc

NC = 2
NS = 16
NW = NC * NS
CH = 512
LN = 16

_F32 = jnp.float32
_I32 = jnp.int32

_SC_PARAMS = dict(
    mesh=plsc.VectorSubcoreMesh(core_axis_name="c", subcore_axis_name="s"),
    compiler_params=pltpu.CompilerParams(use_tc_tiling_on_sc=False, needs_layout_passes=False),
)


def _sc_scatter_scalar(ei3, vals_hbm, zeros1, ones, n_pad, kk):
    rows = ei3.shape[1]
    rw = rows // NW
    nsup = rw // kk
    assert nsup * kk == rw
    npt = n_pad // NS
    gather = vals_hbm is not None

    scratch = [
        pltpu.VMEM((kk, CH), _I32),
        pltpu.VMEM((kk, CH), _F32),
        pltpu.VMEM((npt,), _F32),
        pltpu.VMEM_SHARED((n_pad,), _F32),
        pltpu.SemaphoreType.DMA,
    ]
    if gather:
        scratch = [pltpu.VMEM((kk, CH), _I32),
                   pltpu.VMEM((n_pad,), _F32)] + scratch

    @functools.partial(
        pl.kernel,
        out_type=jax.ShapeDtypeStruct((NC * n_pad,), _F32),
        scratch_types=scratch,
        **_SC_PARAMS,
    )
    def k(*refs):
        if gather:
            ei_h, val_h, z_h, out_h, sidx_v, xtab_v, didx_v, val_v, \
                bb_v, acc, sems = refs
        else:
            ei_h, z_h, ones_h, out_h, didx_v, val_v, bb_v, acc, sems = refs
        c = lax.axis_index("c")
        s = lax.axis_index("s")
        wid = c * NS + s
        moff = pl.multiple_of(s * npt, 8)
        pltpu.sync_copy(z_h.at[pl.ds(moff, npt)], bb_v)
        pltpu.sync_copy(bb_v, acc.at[pl.ds(moff, npt)])
        if gather:
            pltpu.sync_copy(val_h, xtab_v)
        else:
            pltpu.sync_copy(ones_h, val_v)
        plsc.subcore_barrier()
        base = wid * rw

        def body(i, carry):
            r0 = pl.multiple_of(base + i * kk, 8)
            pltpu.sync_copy(ei_h.at[1, pl.ds(r0, kk), :], didx_v)
            if gather:
                pltpu.sync_copy(ei_h.at[0, pl.ds(r0, kk), :], sidx_v)
                for j in range(kk):
                    for q in range(CH // LN):
                        idx16 = sidx_v[j, pl.ds(q * LN, LN)]
                        val_v[j, pl.ds(q * LN, LN)] = \
                            plsc.load_gather(xtab_v, [idx16])
            sd = [pltpu.async_copy(val_v.at[j], acc.at[didx_v.at[j]], sems,
                                   add=True)
                  for j in range(kk)]
            for d_ in sd:
                d_.wait()
            return carry

        lax.fori_loop(0, nsup, body, 0)
        plsc.subcore_barrier()
        ooff = pl.multiple_of(c * n_pad + s * npt, 8)
        pltpu.sync_copy(acc.at[pl.ds(moff, npt)], bb_v)
        pltpu.sync_copy(bb_v, out_h.at[pl.ds(ooff, npt)])

    if gather:
        return k(ei3, vals_hbm, zeros1)
    return k(ei3, zeros1, ones)


def _sc_scatter_rows(ei3, rows0, rows1, zeros2, n_pad, dh, kk):
    rows = ei3.shape[1]
    rt = rows // NS
    nsup = rt // kk
    assert nsup * kk == rt
    npt = n_pad // NS

    @functools.partial(
        pl.kernel,
        out_type=[jax.ShapeDtypeStruct((n_pad, dh), _F32)] * 2,
        scratch_types=[
            pltpu.VMEM((kk, CH), _I32),
            pltpu.VMEM((kk, CH), _I32),
            pltpu.VMEM((kk, CH, dh), _F32),
            pltpu.VMEM((npt, dh), _F32),
            pltpu.VMEM_SHARED((n_pad, dh), _F32),
            pltpu.SemaphoreType.DMA,
            pltpu.SemaphoreType.DMA,
        ],
        **_SC_PARAMS,
    )
    def k(ei_h, rows0_h, rows1_h, z_h, out0_h, out1_h,
          sidx_v, didx_v, rows_v, bb_v, acc, semg, sems):
        c = lax.axis_index("c")
        s = lax.axis_index("s")
        moff = pl.multiple_of(s * npt, 8)
        pltpu.sync_copy(z_h.at[pl.ds(moff, npt), :], bb_v)
        pltpu.sync_copy(bb_v, acc.at[pl.ds(moff, npt), :])
        plsc.subcore_barrier()
        base = s * rt

        def body(i, carry):
            r0 = pl.multiple_of(base + i * kk, 8)
            pltpu.sync_copy(ei_h.at[0, pl.ds(r0, kk), :], sidx_v)
            pltpu.sync_copy(ei_h.at[1, pl.ds(r0, kk), :], didx_v)

            def fire_scatters(gd):
                sd = []
                for j in range(kk):
                    gd[j].wait()
                    sd.append(pltpu.async_copy(rows_v.at[j],
                                               acc.at[didx_v.at[j]],
                                               sems, add=True))
                for d_ in sd:
                    d_.wait()

            @pl.when(c == 0)
            def _():
                fire_scatters([pltpu.async_copy(rows0_h.at[sidx_v.at[j]],
                                                rows_v.at[j], semg)
                               for j in range(kk)])

            @pl.when(c == 1)
            def _():
                fire_scatters([pltpu.async_copy(rows1_h.at[sidx_v.at[j]],
                                                rows_v.at[j], semg)
                               for j in range(kk)])

            return carry

        lax.fori_loop(0, nsup, body, 0)
        plsc.subcore_barrier()
        pltpu.sync_copy(acc.at[pl.ds(moff, npt), :], bb_v)

        @pl.when(c == 0)
        def _():
            pltpu.sync_copy(bb_v, out0_h.at[pl.ds(moff, npt), :])

        @pl.when(c == 1)
        def _():
            pltpu.sync_copy(bb_v, out1_h.at[pl.ds(moff, npt), :])

    return k(ei3, rows0, rows1, zeros2)


def _tc_layer1(svec, xp, dis, w1iT, w1rT, b1T, n_pad, blk):
    d = w1iT.shape[0]
    dh = d // NC

    def body(s_r, x_r, d_r, wi_r, wr_r, b_r, h_r, hh0_r, hh1_r):
        h = jnp.maximum(wi_r[:] * s_r[:] + wr_r[:] * x_r[:] + b_r[:], 0.0)
        h_r[:] = h
        hh = h * d_r[:]
        hh0_r[:] = hh[:dh, :]
        hh1_r[:] = hh[dh:, :]

    row = pl.BlockSpec((1, blk), lambda i: (0, i))
    full = pl.BlockSpec((d, 1), lambda i: (0, 0))
    return pl.pallas_call(
        body,
        grid=(n_pad // blk,),
        in_specs=[row, row, row, full, full, full],
        out_specs=[pl.BlockSpec((d, blk), lambda i: (0, i)),
                   pl.BlockSpec((dh, blk), lambda i: (0, i)),
                   pl.BlockSpec((dh, blk), lambda i: (0, i))],
        out_shape=[jax.ShapeDtypeStruct((d, n_pad), _F32),
                   jax.ShapeDtypeStruct((dh, n_pad), _F32),
                   jax.ShapeDtypeStruct((dh, n_pad), _F32)],
    )(svec, xp, dis, w1iT, w1rT, b1T)


def _tc_layer2(u0T, u1T, dis, h1T, w2iT, w2rT, b2T, fwT, fb, n_pad, blk):
    def body(u0_r, u1_r, d_r, h1_r, wi_r, wr_r, b_r, fw_r, fb_r, y_r):
        g = jnp.concatenate([u0_r[:], u1_r[:]], axis=0) * d_r[:]
        agg = jnp.dot(wi_r[:], g, preferred_element_type=_F32)
        rt = jnp.dot(wr_r[:], h1_r[:], preferred_element_type=_F32)
        out2 = jnp.maximum(agg + rt + b_r[:], 0.0)
        y_r[:] = jnp.dot(fw_r[:], out2, preferred_element_type=_F32) + fb_r[:]

    dd = h1T.shape[0]
    dhh = dd // NC
    do = w2iT.shape[0]
    row = pl.BlockSpec((1, blk), lambda i: (0, i))
    half = pl.BlockSpec((dhh, blk), lambda i: (0, i))
    return pl.pallas_call(
        body,
        grid=(n_pad // blk,),
        in_specs=[half, half, row, pl.BlockSpec((dd, blk), lambda i: (0, i)),
                  pl.BlockSpec((do, dd), lambda i: (0, 0)),
                  pl.BlockSpec((do, dd), lambda i: (0, 0)),
                  pl.BlockSpec((do, 1), lambda i: (0, 0)),
                  pl.BlockSpec((1, do), lambda i: (0, 0)),
                  pl.BlockSpec((1, 1), lambda i: (0, 0))],
        out_specs=pl.BlockSpec((1, blk), lambda i: (0, i)),
        out_shape=jax.ShapeDtypeStruct((1, n_pad), _F32),
    )(u0T, u1T, dis, h1T, w2iT, w2rT, b2T, fwT, fb)


def kernel(x_tmp, edge_index, conv1_init_w, conv1_root_w, conv1_bias,
           conv2_init_w, conv2_root_w, conv2_bias, fc2_w, fc2_b):
    n = x_tmp.shape[0]
    e = edge_index.shape[1]
    n_pad = ((n + 255) // 256) * 256
    blk = n_pad // 4

    kk_s, kk_r = 5, 2
    row_quant = CH * kk_s * NW
    e_pad = ((e + row_quant - 1) // row_quant) * row_quant
    npad_e = e_pad - e
    rows = e_pad // CH
    pad_ar = jnp.arange(npad_e, dtype=_I32)
    pad2 = jnp.stack([pad_ar % n, n + pad_ar % (n_pad - n)])
    ei3 = jnp.concatenate([edge_index, pad2], axis=1).reshape(2, rows, CH)

    zeros1 = jnp.zeros((n_pad,), _F32)
    ones = jnp.ones((kk_s, CH), _F32)

    deg2 = _sc_scatter_scalar(ei3, None, zeros1, ones, n_pad, kk_s)
    deg = deg2[:n_pad] + deg2[n_pad:]
    dis = jnp.where(deg > 0, lax.rsqrt(jnp.maximum(deg, 1e-12)), 0.0)
    xp = jnp.pad(x_tmp[:, 0], (0, n_pad - n))
    xhat = dis * xp

    t2 = _sc_scatter_scalar(ei3, xhat, zeros1, ones, n_pad, kk_s)
    svec = dis * (t2[:n_pad] + t2[n_pad:])

    h1T, hh0T, hh1T = _tc_layer1(
        svec[None, :], xp[None, :], dis[None, :],
        conv1_init_w.reshape(-1, 1), conv1_root_w.reshape(-1, 1),
        conv1_bias[:, None], n_pad, blk)

    dhh = hh0T.shape[0]
    zeros2 = jnp.zeros((n_pad, dhh), _F32)
    u0, u1 = _sc_scatter_rows(ei3, hh0T.T, hh1T.T, zeros2,
                              n_pad, dhh, kk_r)

    yT = _tc_layer2(u0.T, u1.T, dis[None, :], h1T,
                    conv2_init_w.T, conv2_root_w.T, conv2_bias[:, None],
                    fc2_w.T, fc2_b[None, :], n_pad, blk)
    return yT[0, :n][:, None]


# speedup vs baseline: 50.0467x; 1.0640x over previous
import functools

import jax
import jax.numpy as jnp
from jax import lax
from jax.experimental import pallas as pl
from jax.experimental.pallas import tpu as pltpu
from jax.experimental.pallas import tpu_sc as plsc

NC = 2
NS = 16
NW = NC * NS
CH = 512
LN = 16

_F32 = jnp.float32
_I32 = jnp.int32

_SC_PARAMS = dict(
    mesh=plsc.VectorSubcoreMesh(core_axis_name="c", subcore_axis_name="s"),
    compiler_params=pltpu.CompilerParams(use_tc_tiling_on_sc=False, needs_layout_passes=False),
)


def _sc_scatter_scalar(ei3, vals_hbm, zeros1, ones, n_pad, kk):
    rows = ei3.shape[1]
    rw = rows // NW
    nsup = rw // kk
    assert nsup * kk == rw
    npt = n_pad // NS
    gather = vals_hbm is not None

    scratch = [
        pltpu.VMEM((kk, CH), _I32),
        pltpu.VMEM((kk, CH), _F32),
        pltpu.VMEM((npt,), _F32),
        pltpu.VMEM_SHARED((n_pad,), _F32),
        pltpu.SemaphoreType.DMA,
    ]
    if gather:
        scratch = [pltpu.VMEM((kk, CH), _I32),
                   pltpu.VMEM((n_pad,), _F32)] + scratch

    @functools.partial(
        pl.kernel,
        out_type=jax.ShapeDtypeStruct((NC * n_pad,), _F32),
        scratch_types=scratch,
        **_SC_PARAMS,
    )
    def k(*refs):
        if gather:
            ei_h, val_h, z_h, out_h, sidx_v, xtab_v, didx_v, val_v, \
                bb_v, acc, sems = refs
        else:
            ei_h, z_h, ones_h, out_h, didx_v, val_v, bb_v, acc, sems = refs
        c = lax.axis_index("c")
        s = lax.axis_index("s")
        wid = c * NS + s
        moff = pl.multiple_of(s * npt, 8)
        pltpu.sync_copy(z_h.at[pl.ds(moff, npt)], bb_v)
        pltpu.sync_copy(bb_v, acc.at[pl.ds(moff, npt)])
        if gather:
            pltpu.sync_copy(val_h, xtab_v)
        else:
            pltpu.sync_copy(ones_h, val_v)
        plsc.subcore_barrier()
        base = wid * rw

        def body(i, carry):
            r0 = pl.multiple_of(base + i * kk, 8)
            pltpu.sync_copy(ei_h.at[1, pl.ds(r0, kk), :], didx_v)
            if gather:
                pltpu.sync_copy(ei_h.at[0, pl.ds(r0, kk), :], sidx_v)
                for j in range(kk):
                    for q in range(CH // LN):
                        idx16 = sidx_v[j, pl.ds(q * LN, LN)]
                        val_v[j, pl.ds(q * LN, LN)] = \
                            plsc.load_gather(xtab_v, [idx16])
            sd = [pltpu.async_copy(val_v.at[j], acc.at[didx_v.at[j]], sems,
                                   add=True)
                  for j in range(kk)]
            for d_ in sd:
                d_.wait()
            return carry

        lax.fori_loop(0, nsup, body, 0)
        plsc.subcore_barrier()
        ooff = pl.multiple_of(c * n_pad + s * npt, 8)
        pltpu.sync_copy(acc.at[pl.ds(moff, npt)], bb_v)
        pltpu.sync_copy(bb_v, out_h.at[pl.ds(ooff, npt)])

    if gather:
        return k(ei3, vals_hbm, zeros1)
    return k(ei3, zeros1, ones)


def _sc_scatter_rows(ei3, rows0, rows1, zeros2, n_pad, dh, kk):
    rows = ei3.shape[1]
    rt = rows // NS
    nsup = rt // kk
    assert nsup * kk == rt
    npt = n_pad // NS

    @functools.partial(
        pl.kernel,
        out_type=[jax.ShapeDtypeStruct((n_pad, dh), _F32)] * 2,
        scratch_types=[
            pltpu.VMEM((kk, CH), _I32),
            pltpu.VMEM((kk, CH), _I32),
            pltpu.VMEM((kk, CH, dh), _F32),
            pltpu.VMEM((npt, dh), _F32),
            pltpu.VMEM_SHARED((n_pad, dh), _F32),
            pltpu.SemaphoreType.DMA,
            pltpu.SemaphoreType.DMA,
        ],
        **_SC_PARAMS,
    )
    def k(ei_h, rows0_h, rows1_h, z_h, out0_h, out1_h,
          sidx_v, didx_v, rows_v, bb_v, acc, semg, sems):
        c = lax.axis_index("c")
        s = lax.axis_index("s")
        moff = pl.multiple_of(s * npt, 8)
        pltpu.sync_copy(z_h.at[pl.ds(moff, npt), :], bb_v)
        pltpu.sync_copy(bb_v, acc.at[pl.ds(moff, npt), :])
        plsc.subcore_barrier()
        base = s * rt

        def body(i, carry):
            r0 = pl.multiple_of(base + i * kk, 8)
            pltpu.sync_copy(ei_h.at[0, pl.ds(r0, kk), :], sidx_v)
            pltpu.sync_copy(ei_h.at[1, pl.ds(r0, kk), :], didx_v)

            def fire_scatters(gd):
                sd = []
                for j in range(kk):
                    gd[j].wait()
                    sd.append(pltpu.async_copy(rows_v.at[j],
                                               acc.at[didx_v.at[j]],
                                               sems, add=True))
                for d_ in sd:
                    d_.wait()

            @pl.when(c == 0)
            def _():
                fire_scatters([pltpu.async_copy(rows0_h.at[sidx_v.at[j]],
                                                rows_v.at[j], semg)
                               for j in range(kk)])

            @pl.when(c == 1)
            def _():
                fire_scatters([pltpu.async_copy(rows1_h.at[sidx_v.at[j]],
                                                rows_v.at[j], semg)
                               for j in range(kk)])

            return carry

        lax.fori_loop(0, nsup, body, 0)
        plsc.subcore_barrier()
        pltpu.sync_copy(acc.at[pl.ds(moff, npt), :], bb_v)

        @pl.when(c == 0)
        def _():
            pltpu.sync_copy(bb_v, out0_h.at[pl.ds(moff, npt), :])

        @pl.when(c == 1)
        def _():
            pltpu.sync_copy(bb_v, out1_h.at[pl.ds(moff, npt), :])

    return k(ei3, rows0, rows1, zeros2)


def _tc_layer1(svec, xp, dis, w1iT, w1rT, b1T, n_pad, blk):
    d = w1iT.shape[0]
    dh = d // NC

    def body(s_r, x_r, d_r, wi_r, wr_r, b_r, h_r, hh0_r, hh1_r):
        h = jnp.maximum(wi_r[:] * s_r[:] + wr_r[:] * x_r[:] + b_r[:], 0.0)
        h_r[:] = h
        hh = h * d_r[:]
        hh0_r[:] = hh[:dh, :].T
        hh1_r[:] = hh[dh:, :].T

    row = pl.BlockSpec((1, blk), lambda i: (0, i))
    full = pl.BlockSpec((d, 1), lambda i: (0, 0))
    return pl.pallas_call(
        body,
        grid=(n_pad // blk,),
        in_specs=[row, row, row, full, full, full],
        out_specs=[pl.BlockSpec((d, blk), lambda i: (0, i)),
                   pl.BlockSpec((blk, dh), lambda i: (i, 0)),
                   pl.BlockSpec((blk, dh), lambda i: (i, 0))],
        out_shape=[jax.ShapeDtypeStruct((d, n_pad), _F32),
                   jax.ShapeDtypeStruct((n_pad, dh), _F32),
                   jax.ShapeDtypeStruct((n_pad, dh), _F32)],
    )(svec, xp, dis, w1iT, w1rT, b1T)


def _tc_layer2(u0T, u1T, dis, h1T, w2iT, w2rT, b2T, fwT, fb, n_pad, blk):
    def body(u0_r, u1_r, d_r, h1_r, wi_r, wr_r, b_r, fw_r, fb_r, y_r):
        g = jnp.concatenate([u0_r[:].T, u1_r[:].T], axis=0) * d_r[:]
        agg = jnp.dot(wi_r[:], g, preferred_element_type=_F32)
        rt = jnp.dot(wr_r[:], h1_r[:], preferred_element_type=_F32)
        out2 = jnp.maximum(agg + rt + b_r[:], 0.0)
        y_r[:] = jnp.dot(fw_r[:], out2, preferred_element_type=_F32) + fb_r[:]

    dd = h1T.shape[0]
    dhh = dd // NC
    do = w2iT.shape[0]
    row = pl.BlockSpec((1, blk), lambda i: (0, i))
    half = pl.BlockSpec((blk, dhh), lambda i: (i, 0))
    return pl.pallas_call(
        body,
        grid=(n_pad // blk,),
        in_specs=[half, half, row, pl.BlockSpec((dd, blk), lambda i: (0, i)),
                  pl.BlockSpec((do, dd), lambda i: (0, 0)),
                  pl.BlockSpec((do, dd), lambda i: (0, 0)),
                  pl.BlockSpec((do, 1), lambda i: (0, 0)),
                  pl.BlockSpec((1, do), lambda i: (0, 0)),
                  pl.BlockSpec((1, 1), lambda i: (0, 0))],
        out_specs=pl.BlockSpec((1, blk), lambda i: (0, i)),
        out_shape=jax.ShapeDtypeStruct((1, n_pad), _F32),
    )(u0T, u1T, dis, h1T, w2iT, w2rT, b2T, fwT, fb)


def kernel(x_tmp, edge_index, conv1_init_w, conv1_root_w, conv1_bias,
           conv2_init_w, conv2_root_w, conv2_bias, fc2_w, fc2_b):
    n = x_tmp.shape[0]
    e = edge_index.shape[1]
    n_pad = ((n + 255) // 256) * 256
    blk = n_pad // 4

    kk_s, kk_r = 5, 2
    row_quant = CH * kk_s * NW
    e_pad = ((e + row_quant - 1) // row_quant) * row_quant
    npad_e = e_pad - e
    rows = e_pad // CH
    pad_ar = jnp.arange(npad_e, dtype=_I32)
    pad2 = jnp.stack([pad_ar % n, n + pad_ar % (n_pad - n)])
    ei3 = jnp.concatenate([edge_index, pad2], axis=1).reshape(2, rows, CH)

    zeros1 = jnp.zeros((n_pad,), _F32)
    ones = jnp.ones((kk_s, CH), _F32)

    deg2 = _sc_scatter_scalar(ei3, None, zeros1, ones, n_pad, kk_s)
    deg = deg2[:n_pad] + deg2[n_pad:]
    dis = jnp.where(deg > 0, lax.rsqrt(jnp.maximum(deg, 1e-12)), 0.0)
    xp = jnp.pad(x_tmp[:, 0], (0, n_pad - n))
    xhat = dis * xp

    t2 = _sc_scatter_scalar(ei3, xhat, zeros1, ones, n_pad, kk_s)
    svec = dis * (t2[:n_pad] + t2[n_pad:])

    h1T, hh0, hh1 = _tc_layer1(
        svec[None, :], xp[None, :], dis[None, :],
        conv1_init_w.reshape(-1, 1), conv1_root_w.reshape(-1, 1),
        conv1_bias[:, None], n_pad, blk)

    dhh = hh0.shape[1]
    zeros2 = jnp.zeros((n_pad, dhh), _F32)
    u0, u1 = _sc_scatter_rows(ei3, hh0, hh1, zeros2,
                              n_pad, dhh, kk_r)

    yT = _tc_layer2(u0, u1, dis[None, :], h1T,
                    conv2_init_w.T, conv2_root_w.T, conv2_bias[:, None],
                    fc2_w.T, fc2_b[None, :], n_pad, blk)
    return yT[0, :n][:, None]
